# Initial kernel scaffold; baseline (speedup 1.0000x reference)
import jax
import jax.numpy as jnp
from jax.experimental import pallas as pl


def kernel(x, edge_index, batch, W1, b1, W2, b2, W3, b3):
    raise NotImplementedError("write your pallas kernel here")


# speedup vs baseline: 17.1131x; 17.1131x over previous
import functools

import jax
import jax.numpy as jnp
from jax import lax
from jax.experimental import pallas as pl
from jax.experimental.pallas import tpu as pltpu
from jax.experimental.pallas import tpu_sc as plsc

N = 100000
E = 1600000
H = 64
G = 64
IN = 4

NPAD = 100352
ZROWS = NPAD - N
TSLICE = NPAD // 16

EPAD = 1638400
EROWS = EPAD // 128
WROWS = EROWS // 32
BLKR = 8
NBLK = WROWS // BLKR

BR = 1792
NBR = NPAD // BR

_mesh = plsc.VectorSubcoreMesh(core_axis_name="c", subcore_axis_name="s")
_sc_params = pltpu.CompilerParams(use_tc_tiling_on_sc=False)


def _wid():
    return lax.axis_index("s") * 2 + lax.axis_index("c")


@functools.partial(
    pl.kernel,
    out_type=jax.ShapeDtypeStruct((2 * NPAD, 16), jnp.float32),
    mesh=_mesh,
    compiler_params=_sc_params,
    scratch_types=[
        pltpu.VMEM((BLKR, 128), jnp.int32),
        pltpu.VMEM((128, 16), jnp.float32),
        pltpu.VMEM_SHARED((NPAD, 16), jnp.float32),
    ],
)
def _deg_kernel(dstr_h, erow_h, zeros_h, out_h, didx_v, erow_v, acc_sh):
    cid = lax.axis_index("c")
    sid = lax.axis_index("s")
    wid = _wid()
    pltpu.sync_copy(erow_h, erow_v)
    pltpu.sync_copy(zeros_h, acc_sh.at[pl.ds(sid * TSLICE, TSLICE)])
    plsc.subcore_barrier()

    def blk(k, carry):
        base = wid * WROWS + k * BLKR
        pltpu.sync_copy(dstr_h.at[pl.ds(base, BLKR)], didx_v)
        for j in range(BLKR):
            pltpu.sync_copy(erow_v, acc_sh.at[didx_v.at[j]], add=True)
        return carry

    lax.fori_loop(0, NBLK, blk, 0)
    plsc.subcore_barrier()
    pltpu.sync_copy(
        acc_sh.at[pl.ds(sid * TSLICE, TSLICE)],
        out_h.at[pl.ds(cid * NPAD + sid * TSLICE, TSLICE)],
    )


def _make_agg(F, nsrc):
    @functools.partial(
        pl.kernel,
        out_type=jax.ShapeDtypeStruct((nsrc * 2 * NPAD, F), jnp.float32),
        mesh=_mesh,
        compiler_params=_sc_params,
        scratch_types=[
            pltpu.VMEM((BLKR, 128), jnp.int32),
            pltpu.VMEM((BLKR, 128), jnp.int32),
            pltpu.VMEM((BLKR * 128, F), jnp.float32),
            pltpu.VMEM_SHARED((NPAD, F), jnp.float32),
            pltpu.SemaphoreType.DMA,
        ],
    )
    def _agg(srcr_h, dstr_h, *rest):
        srcs = rest[:nsrc]
        zeros_h = rest[nsrc]
        out_h = rest[nsrc + 1]
        sidx_v, didx_v, rows_v, acc_sh, sem = rest[nsrc + 2:]
        cid = lax.axis_index("c")
        sid = lax.axis_index("s")
        wid = _wid()

        for p in range(nsrc):
            pltpu.sync_copy(zeros_h, acc_sh.at[pl.ds(sid * TSLICE, TSLICE)])
            plsc.subcore_barrier()

            def blk(k, carry):
                base = wid * WROWS + k * BLKR
                pltpu.sync_copy(srcr_h.at[pl.ds(base, BLKR)], sidx_v)
                pltpu.sync_copy(dstr_h.at[pl.ds(base, BLKR)], didx_v)
                descs = [
                    pltpu.async_copy(
                        srcs[p].at[sidx_v.at[j]],
                        rows_v.at[pl.ds(j * 128, 128)],
                        sem,
                    )
                    for j in range(BLKR)
                ]
                for d in descs:
                    d.wait()
                for j in range(BLKR):
                    pltpu.sync_copy(
                        rows_v.at[pl.ds(j * 128, 128)],
                        acc_sh.at[didx_v.at[j]],
                        add=True,
                    )
                return carry

            lax.fori_loop(0, NBLK, blk, 0)
            plsc.subcore_barrier()
            pltpu.sync_copy(
                acc_sh.at[pl.ds(sid * TSLICE, TSLICE)],
                out_h.at[pl.ds((p * 2 + cid) * NPAD + sid * TSLICE, TSLICE)],
            )

    return _agg


_agg_in = _make_agg(16, 1)
_agg_h = _make_agg(16, 4)


def _prep_body(deg0, deg1, x, dinv_o, xs_o):
    deg = deg0[...] + deg1[...] + 1.0
    dinv = lax.rsqrt(deg)
    dinv_o[...] = dinv
    xs_o[...] = jnp.concatenate(
        [x[...] * dinv, jnp.zeros((BR, 16 - IN), jnp.float32)], axis=1)


def _prep_call(deg0, deg1, xpad):
    col1 = pl.BlockSpec((BR, 1), lambda i: (i, 0))
    return pl.pallas_call(
        _prep_body,
        grid=(NBR,),
        in_specs=[col1, col1, pl.BlockSpec((BR, IN), lambda i: (i, 0))],
        out_specs=[col1, pl.BlockSpec((BR, 16), lambda i: (i, 0))],
        out_shape=[
            jax.ShapeDtypeStruct((NPAD, 1), jnp.float32),
            jax.ShapeDtypeStruct((NPAD, 16), jnp.float32),
        ],
    )(deg0, deg1, xpad)


def _dense1_body(a0, a1, xs, dinv, W1, b1, o0, o1, o2, o3):
    a = (a0[...] + a1[...] + xs[...]) * dinv[...]
    z = lax.dot_general(a, W1[...], (((1,), (0,)), ((), ())),
                        preferred_element_type=jnp.float32) + b1[...]
    h = jnp.maximum(z, 0.0) * dinv[...]
    o0[...] = h[:, 0:16]
    o1[...] = h[:, 16:32]
    o2[...] = h[:, 32:48]
    o3[...] = h[:, 48:64]


def _dense1_call(a0, a1, xs, dinv, W1p, b1):
    row = pl.BlockSpec((BR, 16), lambda i: (i, 0))
    return pl.pallas_call(
        _dense1_body,
        grid=(NBR,),
        in_specs=[
            row, row, row,
            pl.BlockSpec((BR, 1), lambda i: (i, 0)),
            pl.BlockSpec((16, H), lambda i: (0, 0)),
            pl.BlockSpec((1, H), lambda i: (0, 0)),
        ],
        out_specs=[pl.BlockSpec((BR, 16), lambda i: (i, 0))] * 4,
        out_shape=[jax.ShapeDtypeStruct((NPAD, 16), jnp.float32)] * 4,
    )(a0, a1, xs, dinv, W1p, b1)


def _dense2_body(a00, a01, a10, a11, a20, a21, a30, a31,
                 h0, h1, h2, h3, dinv, bat, W2, b2, W3, b3,
                 out_o, acc_v, cnt_v):
    g = pl.program_id(0)
    a2 = jnp.concatenate(
        [a00[...] + a01[...] + h0[...],
         a10[...] + a11[...] + h1[...],
         a20[...] + a21[...] + h2[...],
         a30[...] + a31[...] + h3[...]], axis=1)
    z = lax.dot_general(a2 * dinv[...], W2[...], (((1,), (0,)), ((), ())),
                        preferred_element_type=jnp.float32) + b2[...]
    hh = jnp.maximum(z, 0.0)
    oneh = (bat[...] == lax.broadcasted_iota(jnp.int32, (1, G), 1)
            ).astype(jnp.float32)
    pa = lax.dot_general(oneh, hh, (((0,), (0,)), ((), ())),
                         preferred_element_type=jnp.float32)
    pc = lax.dot_general(oneh, jnp.ones((BR, 1), jnp.float32),
                         (((0,), (0,)), ((), ())),
                         preferred_element_type=jnp.float32)

    @pl.when(g == 0)
    def _():
        acc_v[...] = jnp.zeros((G, H), jnp.float32)
        cnt_v[...] = jnp.zeros((G, 1), jnp.float32)

    acc_v[...] += pa
    cnt_v[...] += pc

    @pl.when(g == NBR - 1)
    def _():
        mean = acc_v[...] / jnp.maximum(cnt_v[...], 1.0)
        out_o[...] = lax.dot_general(mean, W3[...], (((1,), (0,)), ((), ())),
                                     preferred_element_type=jnp.float32) + b3[...]


def _dense2_call(aggs, hs, dinv, batchp, W2, b2, W3, b3):
    row16 = pl.BlockSpec((BR, 16), lambda i: (i, 0))
    return pl.pallas_call(
        _dense2_body,
        grid=(NBR,),
        in_specs=(
            [row16] * 8 + [row16] * 4
            + [pl.BlockSpec((BR, 1), lambda i: (i, 0)),
               pl.BlockSpec((BR, 1), lambda i: (i, 0)),
               pl.BlockSpec((H, H), lambda i: (0, 0)),
               pl.BlockSpec((1, H), lambda i: (0, 0)),
               pl.BlockSpec((H, 1), lambda i: (0, 0)),
               pl.BlockSpec((1, 1), lambda i: (0, 0))]
        ),
        out_specs=pl.BlockSpec((G, 1), lambda i: (0, 0)),
        out_shape=jax.ShapeDtypeStruct((G, 1), jnp.float32),
        scratch_shapes=[
            pltpu.VMEM((G, H), jnp.float32),
            pltpu.VMEM((G, 1), jnp.float32),
        ],
    )(*aggs, *hs, dinv, batchp, W2, b2, W3, b3)


def kernel(x, edge_index, batch, W1, b1, W2, b2, W3, b3):
    pad = N + (jnp.arange(EPAD - E, dtype=jnp.int32) % ZROWS)
    srcr = jnp.concatenate([edge_index[0], pad]).reshape(EROWS, 128)
    dstr = jnp.concatenate([edge_index[1], pad]).reshape(EROWS, 128)

    zeros16 = jnp.zeros((TSLICE, 16), jnp.float32)
    erow = jnp.zeros((128, 16), jnp.float32).at[:, 0].set(1.0)
    degflat = _deg_kernel(dstr, erow, zeros16)
    deg0 = degflat[:NPAD, 0:1]
    deg1 = degflat[NPAD:, 0:1]

    xpad = jnp.pad(x, ((0, NPAD - N), (0, 0)))
    dinv, xs = _prep_call(deg0, deg1, xpad)

    agg1 = _agg_in(srcr, dstr, xs, zeros16)
    W1p = jnp.pad(W1, ((0, 16 - IN), (0, 0)))
    hs = _dense1_call(agg1[:NPAD], agg1[NPAD:], xs, dinv, W1p, b1.reshape(1, H))

    agg2 = _agg_h(srcr, dstr, *hs, zeros16)
    a2 = [agg2[k * NPAD:(k + 1) * NPAD] for k in range(8)]

    batchp = jnp.pad(batch, (0, NPAD - N), constant_values=G)[:, None]
    out = _dense2_call(a2, hs, dinv, batchp, W2, b2.reshape(1, H),
                       W3, b3.reshape(1, 1))
    return out


# speedup vs baseline: 17.3859x; 1.0159x over previous
import functools

import jax
import jax.numpy as jnp
from jax import lax
from jax.experimental import pallas as pl
from jax.experimental.pallas import tpu as pltpu
from jax.experimental.pallas import tpu_sc as plsc

N = 100000
E = 1600000
H = 64
G = 64
IN = 4

NPAD = 100352
ZROWS = NPAD - N
TSLICE = NPAD // 16

EPAD = 1638400
EROWS = EPAD // 128
WROWS = EROWS // 32
BLKR = 8
NBLK = WROWS // BLKR

BR = 1792
NBR = NPAD // BR

_mesh = plsc.VectorSubcoreMesh(core_axis_name="c", subcore_axis_name="s")
_sc_params = pltpu.CompilerParams(use_tc_tiling_on_sc=False)


def _wid():
    return lax.axis_index("s") * 2 + lax.axis_index("c")


@functools.partial(
    pl.kernel,
    out_type=jax.ShapeDtypeStruct((2 * NPAD, 16), jnp.float32),
    mesh=_mesh,
    compiler_params=_sc_params,
    scratch_types=[
        pltpu.VMEM((BLKR, 128), jnp.int32),
        pltpu.VMEM((128, 16), jnp.float32),
        pltpu.VMEM_SHARED((NPAD, 16), jnp.float32),
    ],
)
def _deg_kernel(dstr_h, erow_h, zeros_h, out_h, didx_v, erow_v, acc_sh):
    cid = lax.axis_index("c")
    sid = lax.axis_index("s")
    wid = _wid()
    pltpu.sync_copy(erow_h, erow_v)
    pltpu.sync_copy(zeros_h, acc_sh.at[pl.ds(sid * TSLICE, TSLICE)])
    plsc.subcore_barrier()

    def blk(k, carry):
        base = wid * WROWS + k * BLKR
        pltpu.sync_copy(dstr_h.at[pl.ds(base, BLKR)], didx_v)
        for j in range(BLKR):
            pltpu.sync_copy(erow_v, acc_sh.at[didx_v.at[j]], add=True)
        return carry

    lax.fori_loop(0, NBLK, blk, 0)
    plsc.subcore_barrier()
    pltpu.sync_copy(
        acc_sh.at[pl.ds(sid * TSLICE, TSLICE)],
        out_h.at[pl.ds(cid * NPAD + sid * TSLICE, TSLICE)],
    )


def _make_agg(F, nsrc):
    @functools.partial(
        pl.kernel,
        out_type=jax.ShapeDtypeStruct((nsrc * 2 * NPAD, F), jnp.float32),
        mesh=_mesh,
        compiler_params=_sc_params,
        scratch_types=[
            pltpu.VMEM((BLKR, 128), jnp.int32),
            pltpu.VMEM((BLKR, 128), jnp.int32),
            pltpu.VMEM((BLKR * 128, F), jnp.float32),
            pltpu.VMEM_SHARED((NPAD, F), jnp.float32),
            pltpu.SemaphoreType.DMA,
        ],
    )
    def _agg(srcr_h, dstr_h, *rest):
        srcs = rest[:nsrc]
        zeros_h = rest[nsrc]
        out_h = rest[nsrc + 1]
        sidx_v, didx_v, rows_v, acc_sh, sem = rest[nsrc + 2:]
        cid = lax.axis_index("c")
        sid = lax.axis_index("s")
        wid = _wid()

        for p in range(nsrc):
            pltpu.sync_copy(zeros_h, acc_sh.at[pl.ds(sid * TSLICE, TSLICE)])
            plsc.subcore_barrier()

            def blk(k, carry):
                base = wid * WROWS + k * BLKR
                pltpu.sync_copy(srcr_h.at[pl.ds(base, BLKR)], sidx_v)
                pltpu.sync_copy(dstr_h.at[pl.ds(base, BLKR)], didx_v)
                descs = [
                    pltpu.async_copy(
                        srcs[p].at[sidx_v.at[j]],
                        rows_v.at[pl.ds(j * 128, 128)],
                        sem,
                    )
                    for j in range(BLKR)
                ]
                for d in descs:
                    d.wait()
                for j in range(BLKR):
                    pltpu.sync_copy(
                        rows_v.at[pl.ds(j * 128, 128)],
                        acc_sh.at[didx_v.at[j]],
                        add=True,
                    )
                return carry

            lax.fori_loop(0, NBLK, blk, 0)
            plsc.subcore_barrier()
            pltpu.sync_copy(
                acc_sh.at[pl.ds(sid * TSLICE, TSLICE)],
                out_h.at[pl.ds((p * 2 + cid) * NPAD + sid * TSLICE, TSLICE)],
            )

    return _agg


_agg_in = _make_agg(16, 1)
_agg_h = _make_agg(16, 4)


NP8 = NPAD // 8
BRP = 1568
NBP = NP8 // BRP

_CON = (((1,), (0,)), ((), ()))


def _prep_body(deg0, deg1, x, dinv_o, xs_o):
    dinv = lax.rsqrt(deg0[...] + deg1[...] + 1.0)
    dinv_o[...] = dinv
    xs_o[...] = x[...] * dinv


def _prep_call(deg0p, deg1p, xp):
    blk = pl.BlockSpec((BRP, 128), lambda i: (i, 0))
    return pl.pallas_call(
        _prep_body,
        grid=(NBP,),
        in_specs=[blk, blk, blk],
        out_specs=[blk, blk],
        out_shape=[jax.ShapeDtypeStruct((NP8, 128), jnp.float32)] * 2,
    )(deg0p, deg1p, xp)


def _dense1_body(a0, a1, xs, dinv, w0, w1, w2, w3, c0, c1, c2, c3,
                 o0, o1, o2, o3):
    dv = dinv[...]
    a = (a0[...] + a1[...] + xs[...]) * dv
    for w, c, o in ((w0, c0, o0), (w1, c1, o1), (w2, c2, o2), (w3, c3, o3)):
        z = lax.dot_general(a, w[...], _CON,
                            preferred_element_type=jnp.float32) + c[...]
        o[...] = jnp.maximum(z, 0.0) * dv


def _dense1_call(a0p, a1p, xsp, dinvp, Ws, cs):
    blk = pl.BlockSpec((BRP, 128), lambda i: (i, 0))
    wblk = pl.BlockSpec((128, 128), lambda i: (0, 0))
    cblk = pl.BlockSpec((1, 128), lambda i: (0, 0))
    return pl.pallas_call(
        _dense1_body,
        grid=(NBP,),
        in_specs=[blk] * 4 + [wblk] * 4 + [cblk] * 4,
        out_specs=[blk] * 4,
        out_shape=[jax.ShapeDtypeStruct((NP8, 128), jnp.float32)] * 4,
    )(a0p, a1p, xsp, dinvp, *Ws, *cs)


def _dense2a_body(a00, a01, a10, a11, a20, a21, a30, a31,
                  h0, h1, h2, h3, dinv, w0, w1, w2, w3, bt2, wt3, o):
    dv = dinv[...]
    z = bt2[...]
    for ac, bc, hc, w in ((a00, a01, h0, w0), (a10, a11, h1, w1),
                          (a20, a21, h2, w2), (a30, a31, h3, w3)):
        a = (ac[...] + bc[...] + hc[...]) * dv
        z = z + lax.dot_general(a, w[...], _CON,
                                preferred_element_type=jnp.float32)
    hh = jnp.maximum(z, 0.0)
    o[...] = lax.dot_general(hh, wt3[...], _CON,
                             preferred_element_type=jnp.float32)


def _dense2a_call(a2p, hps, dinvp, Ws, bt2, Wt3):
    blk = pl.BlockSpec((BRP, 128), lambda i: (i, 0))
    return pl.pallas_call(
        _dense2a_body,
        grid=(NBP,),
        in_specs=(
            [blk] * 13
            + [pl.BlockSpec((128, 8 * H), lambda i: (0, 0))] * 4
            + [pl.BlockSpec((1, 8 * H), lambda i: (0, 0)),
               pl.BlockSpec((8 * H, 8), lambda i: (0, 0))]
        ),
        out_specs=pl.BlockSpec((BRP, 8), lambda i: (i, 0)),
        out_shape=jax.ShapeDtypeStruct((NP8, 8), jnp.float32),
    )(*a2p, *hps, dinvp, *Ws, bt2, Wt3)


def _dense2b_body(s, bat, b3, out_o, accs_v, accc_v):
    g = pl.program_id(0)
    oneh = (bat[...] == lax.broadcasted_iota(jnp.int32, (1, G), 1)
            ).astype(jnp.float32)
    ps = lax.dot_general(oneh, s[...], (((0,), (0,)), ((), ())),
                         preferred_element_type=jnp.float32)
    pc = lax.dot_general(oneh, jnp.ones((BR, 1), jnp.float32),
                         (((0,), (0,)), ((), ())),
                         preferred_element_type=jnp.float32)

    @pl.when(g == 0)
    def _():
        accs_v[...] = jnp.zeros((G, 1), jnp.float32)
        accc_v[...] = jnp.zeros((G, 1), jnp.float32)

    accs_v[...] += ps
    accc_v[...] += pc

    @pl.when(g == NBR - 1)
    def _():
        out_o[...] = accs_v[...] / jnp.maximum(accc_v[...], 1.0) + b3[...]


def _dense2b_call(s, batchp, b3):
    col = pl.BlockSpec((BR, 1), lambda i: (i, 0))
    return pl.pallas_call(
        _dense2b_body,
        grid=(NBR,),
        in_specs=[col, col, pl.BlockSpec((1, 1), lambda i: (0, 0))],
        out_specs=pl.BlockSpec((G, 1), lambda i: (0, 0)),
        out_shape=jax.ShapeDtypeStruct((G, 1), jnp.float32),
        scratch_shapes=[
            pltpu.VMEM((G, 1), jnp.float32),
            pltpu.VMEM((G, 1), jnp.float32),
        ],
    )(s, batchp, b3)


def kernel(x, edge_index, batch, W1, b1, W2, b2, W3, b3):
    pad = N + (jnp.arange(EPAD - E, dtype=jnp.int32) % ZROWS)
    srcr = jnp.concatenate([edge_index[0], pad]).reshape(EROWS, 128)
    dstr = jnp.concatenate([edge_index[1], pad]).reshape(EROWS, 128)

    zeros16 = jnp.zeros((TSLICE, 16), jnp.float32)
    erow = jnp.ones((128, 16), jnp.float32)
    degflat = _deg_kernel(dstr, erow, zeros16)
    deg0p = degflat[:NPAD].reshape(NP8, 128)
    deg1p = degflat[NPAD:].reshape(NP8, 128)

    xp = jnp.pad(x, ((0, NPAD - N), (0, 16 - IN))).reshape(NP8, 128)
    dinvp, xsp = _prep_call(deg0p, deg1p, xp)

    agg1 = _agg_in(srcr, dstr, xsp.reshape(NPAD, 16), zeros16)
    a0p = agg1[:NPAD].reshape(NP8, 128)
    a1p = agg1[NPAD:].reshape(NP8, 128)

    eye8 = jnp.eye(8, dtype=jnp.float32)
    W1p = jnp.pad(W1, ((0, 16 - IN), (0, 0)))
    Ws1 = [jnp.kron(eye8, W1p[:, 16 * p:16 * p + 16]) for p in range(4)]
    cs1 = [jnp.tile(b1[16 * p:16 * p + 16], 8).reshape(1, 128) for p in range(4)]
    hps = _dense1_call(a0p, a1p, xsp, dinvp, Ws1, cs1)

    agg2 = _agg_h(srcr, dstr, *[h.reshape(NPAD, 16) for h in hps], zeros16)
    a2p = [agg2[k * NPAD:(k + 1) * NPAD].reshape(NP8, 128) for k in range(8)]

    Ws2 = [jnp.kron(eye8, W2[16 * p:16 * p + 16, :]) for p in range(4)]
    bt2 = jnp.tile(b2, 8).reshape(1, 8 * H)
    Wt3 = jnp.kron(eye8, W3)
    sp = _dense2a_call(a2p, hps, dinvp, Ws2, bt2, Wt3)

    s = sp.reshape(NPAD, 1)
    batchp = jnp.pad(batch, (0, NPAD - N), constant_values=G)[:, None]
    return _dense2b_call(s, batchp, b3.reshape(1, 1))


# speedup vs baseline: 31.6024x; 1.8177x over previous
import functools

import jax
import jax.numpy as jnp
from jax import lax
from jax.experimental import pallas as pl
from jax.experimental.pallas import tpu as pltpu
from jax.experimental.pallas import tpu_sc as plsc

N = 100000
E = 1600000
H = 64
G = 64
IN = 4

NPAD = 100352
ZROWS = NPAD - N
TSLICE = NPAD // 16

EPAD = 1638400
EROWS = EPAD // 128
WROWS = EROWS // 32
BLKR = 8
NBLK = WROWS // BLKR

BR = 1792
NBR = NPAD // BR

_mesh = plsc.VectorSubcoreMesh(core_axis_name="c", subcore_axis_name="s")
_sc_params = pltpu.CompilerParams(use_tc_tiling_on_sc=False)


def _wid():
    return lax.axis_index("s") * 2 + lax.axis_index("c")


@functools.partial(
    pl.kernel,
    out_type=jax.ShapeDtypeStruct((2 * NPAD, 16), jnp.float32),
    mesh=_mesh,
    compiler_params=_sc_params,
    scratch_types=[
        pltpu.VMEM((BLKR, 128), jnp.int32),
        pltpu.VMEM((128, 16), jnp.float32),
        pltpu.VMEM_SHARED((NPAD, 16), jnp.float32),
    ],
)
def _deg_kernel(dstr_h, erow_h, zeros_h, out_h, didx_v, erow_v, acc_sh):
    cid = lax.axis_index("c")
    sid = lax.axis_index("s")
    wid = _wid()
    pltpu.sync_copy(erow_h, erow_v)
    pltpu.sync_copy(zeros_h, acc_sh.at[pl.ds(sid * TSLICE, TSLICE)])
    plsc.subcore_barrier()

    def blk(k, carry):
        base = wid * WROWS + k * BLKR
        pltpu.sync_copy(dstr_h.at[pl.ds(base, BLKR)], didx_v)
        for j in range(BLKR):
            pltpu.sync_copy(erow_v, acc_sh.at[didx_v.at[j]], add=True)
        return carry

    lax.fori_loop(0, NBLK, blk, 0)
    plsc.subcore_barrier()
    pltpu.sync_copy(
        acc_sh.at[pl.ds(sid * TSLICE, TSLICE)],
        out_h.at[pl.ds(cid * NPAD + sid * TSLICE, TSLICE)],
    )


def _make_agg(F, nsrc):
    @functools.partial(
        pl.kernel,
        out_type=jax.ShapeDtypeStruct((nsrc * 2 * NPAD, F), jnp.float32),
        mesh=_mesh,
        compiler_params=_sc_params,
        scratch_types=[
            pltpu.VMEM((BLKR, 128), jnp.int32),
            pltpu.VMEM((BLKR, 128), jnp.int32),
            pltpu.VMEM((BLKR * 128, F), jnp.float32),
            pltpu.VMEM_SHARED((NPAD, F), jnp.float32),
            pltpu.SemaphoreType.DMA,
        ],
    )
    def _agg(srcr_h, dstr_h, src_h, zeros_h, out_h,
             sidx_v, didx_v, rows_v, acc_sh, sem):
        cid = lax.axis_index("c")
        sid = lax.axis_index("s")
        wid = _wid()

        for p in range(nsrc):
            src_p = src_h.at[pl.ds(p * NPAD, NPAD)]
            pltpu.sync_copy(zeros_h, acc_sh.at[pl.ds(sid * TSLICE, TSLICE)])
            plsc.subcore_barrier()

            def blk(k, carry):
                base = wid * WROWS + k * BLKR
                pltpu.sync_copy(srcr_h.at[pl.ds(base, BLKR)], sidx_v)
                pltpu.sync_copy(dstr_h.at[pl.ds(base, BLKR)], didx_v)
                descs = [
                    pltpu.async_copy(
                        src_p.at[sidx_v.at[j]],
                        rows_v.at[pl.ds(j * 128, 128)],
                        sem,
                    )
                    for j in range(BLKR)
                ]
                for d in descs:
                    d.wait()
                for j in range(BLKR):
                    pltpu.sync_copy(
                        rows_v.at[pl.ds(j * 128, 128)],
                        acc_sh.at[didx_v.at[j]],
                        add=True,
                    )
                return carry

            lax.fori_loop(0, NBLK, blk, 0)
            plsc.subcore_barrier()
            pltpu.sync_copy(
                acc_sh.at[pl.ds(sid * TSLICE, TSLICE)],
                out_h.at[pl.ds((p * 2 + cid) * NPAD + sid * TSLICE, TSLICE)],
            )

    return _agg


_agg_in = _make_agg(16, 1)
_agg_h = _make_agg(16, 4)


NP8 = NPAD // 8
BRP = 1568
NBP = NP8 // BRP

_CON = (((1,), (0,)), ((), ()))


def _prep_body(deg0, deg1, x, dinv_o, xs_o):
    dinv = lax.rsqrt(deg0[...] + deg1[...] + 1.0)
    dinv_o[...] = dinv
    xs_o[...] = x[...] * dinv


def _prep_call(degp, xp):
    blk = pl.BlockSpec((BRP, 128), lambda i: (i, 0))
    return pl.pallas_call(
        _prep_body,
        grid=(NBP,),
        in_specs=[blk, pl.BlockSpec((BRP, 128), lambda i: (NBP + i, 0)), blk],
        out_specs=[blk, blk],
        out_shape=[jax.ShapeDtypeStruct((NP8, 128), jnp.float32)] * 2,
    )(degp, degp, xp)


def _dense1_body(a0, a1, xs, dinv, w, c, o):
    dv = dinv[...]
    a = (a0[...] + a1[...] + xs[...]) * dv
    z = lax.dot_general(a, w[...], _CON,
                        preferred_element_type=jnp.float32) + c[0]
    o[...] = jnp.maximum(z, 0.0) * dv


def _dense1_call(agg1p, xsp, dinvp, Ws1s, cs1s):
    blk = pl.BlockSpec((BRP, 128), lambda p, i: (i, 0))
    return pl.pallas_call(
        _dense1_body,
        grid=(4, NBP),
        in_specs=[
            blk,
            pl.BlockSpec((BRP, 128), lambda p, i: (NBP + i, 0)),
            blk, blk,
            pl.BlockSpec((128, 128), lambda p, i: (p, 0)),
            pl.BlockSpec((1, 1, 128), lambda p, i: (p, 0, 0)),
        ],
        out_specs=pl.BlockSpec((BRP, 128), lambda p, i: (p * NBP + i, 0)),
        out_shape=jax.ShapeDtypeStruct((4 * NP8, 128), jnp.float32),
    )(agg1p, agg1p, xsp, dinvp, Ws1s, cs1s)


def _dense2a_body(a00, a01, a10, a11, a20, a21, a30, a31,
                  h0, h1, h2, h3, dinv, w0, w1, w2, w3, bt2, wt3, o):
    dv = dinv[...]
    z = bt2[...]
    for ac, bc, hc, w in ((a00, a01, h0, w0), (a10, a11, h1, w1),
                          (a20, a21, h2, w2), (a30, a31, h3, w3)):
        a = (ac[...] + bc[...] + hc[...]) * dv
        z = z + lax.dot_general(a, w[...], _CON,
                                preferred_element_type=jnp.float32)
    hh = jnp.maximum(z, 0.0)
    o[...] = lax.dot_general(hh, wt3[...], _CON,
                             preferred_element_type=jnp.float32)


def _dense2a_call(agg2p, hstk, dinvp, Ws, bt2, Wt3):
    def off(k):
        return pl.BlockSpec((BRP, 128), lambda i, k=k: (k * NBP + i, 0))
    return pl.pallas_call(
        _dense2a_body,
        grid=(NBP,),
        in_specs=(
            [off(k) for k in range(8)]
            + [off(p) for p in range(4)]
            + [pl.BlockSpec((BRP, 128), lambda i: (i, 0))]
            + [pl.BlockSpec((128, 8 * H), lambda i: (0, 0))] * 4
            + [pl.BlockSpec((1, 8 * H), lambda i: (0, 0)),
               pl.BlockSpec((8 * H, 8), lambda i: (0, 0))]
        ),
        out_specs=pl.BlockSpec((BRP, 8), lambda i: (i, 0)),
        out_shape=jax.ShapeDtypeStruct((NP8, 8), jnp.float32),
    )(*([agg2p] * 8), *([hstk] * 4), dinvp, *Ws, bt2, Wt3)


BR8 = 1568
NB8 = NP8 // BR8


def _dense2b_body(s8, bat8, b3, out_o, accs_v, accc_v):
    g = pl.program_id(0)
    iota = lax.broadcasted_iota(jnp.int32, (1, G), 1)
    ps = jnp.zeros((G, 1), jnp.float32)
    pc = jnp.zeros((G, 1), jnp.float32)
    ones = jnp.ones((BR8, 1), jnp.float32)
    for j in range(8):
        ohj = (bat8[:, j:j + 1] == iota).astype(jnp.float32)
        ps = ps + lax.dot_general(ohj, s8[:, j:j + 1],
                                  (((0,), (0,)), ((), ())),
                                  preferred_element_type=jnp.float32)
        pc = pc + lax.dot_general(ohj, ones, (((0,), (0,)), ((), ())),
                                  preferred_element_type=jnp.float32)

    @pl.when(g == 0)
    def _():
        accs_v[...] = jnp.zeros((G, 1), jnp.float32)
        accc_v[...] = jnp.zeros((G, 1), jnp.float32)

    accs_v[...] += ps
    accc_v[...] += pc

    @pl.when(g == NB8 - 1)
    def _():
        out_o[...] = accs_v[...] / jnp.maximum(accc_v[...], 1.0) + b3[...]


def _dense2b_call(s8, bat8, b3):
    blk = pl.BlockSpec((BR8, 8), lambda i: (i, 0))
    return pl.pallas_call(
        _dense2b_body,
        grid=(NB8,),
        in_specs=[blk, blk, pl.BlockSpec((1, 1), lambda i: (0, 0))],
        out_specs=pl.BlockSpec((G, 1), lambda i: (0, 0)),
        out_shape=jax.ShapeDtypeStruct((G, 1), jnp.float32),
        scratch_shapes=[
            pltpu.VMEM((G, 1), jnp.float32),
            pltpu.VMEM((G, 1), jnp.float32),
        ],
    )(s8, bat8, b3)


def kernel(x, edge_index, batch, W1, b1, W2, b2, W3, b3):
    pad = N + (jnp.arange(EPAD - E, dtype=jnp.int32) % ZROWS)
    srcr = jnp.concatenate([edge_index[0], pad]).reshape(EROWS, 128)
    dstr = jnp.concatenate([edge_index[1], pad]).reshape(EROWS, 128)

    zeros16 = jnp.zeros((TSLICE, 16), jnp.float32)
    erow = jnp.ones((128, 16), jnp.float32)
    degp = _deg_kernel(dstr, erow, zeros16).reshape(2 * NP8, 128)

    xp = jnp.pad(x, ((0, NPAD - N), (0, 16 - IN))).reshape(NP8, 128)
    dinvp, xsp = _prep_call(degp, xp)

    agg1p = _agg_in(srcr, dstr, xsp.reshape(NPAD, 16),
                    zeros16).reshape(2 * NP8, 128)

    eye8 = jnp.eye(8, dtype=jnp.float32)
    W1p = jnp.pad(W1, ((0, 16 - IN), (0, 0)))
    Ws1s = jnp.concatenate(
        [jnp.kron(eye8, W1p[:, 16 * p:16 * p + 16]) for p in range(4)], axis=0)
    cs1s = jnp.stack(
        [jnp.tile(b1[16 * p:16 * p + 16], 8) for p in range(4)],
        axis=0).reshape(4, 1, 128)
    hstk = _dense1_call(agg1p, xsp, dinvp, Ws1s, cs1s)

    agg2p = _agg_h(srcr, dstr, hstk.reshape(4 * NPAD, 16),
                   zeros16).reshape(8 * NP8, 128)

    Ws2 = [jnp.kron(eye8, W2[16 * p:16 * p + 16, :]) for p in range(4)]
    bt2 = jnp.tile(b2, 8).reshape(1, 8 * H)
    Wt3 = jnp.kron(eye8, W3)
    sp = _dense2a_call(agg2p, hstk, dinvp, Ws2, bt2, Wt3)

    bat8 = jnp.pad(batch, (0, NPAD - N), constant_values=G).reshape(NP8, 8)
    return _dense2b_call(sp, bat8, b3.reshape(1, 1))


# speedup vs baseline: 35.4146x; 1.1206x over previous
import functools

import jax
import jax.numpy as jnp
from jax import lax
from jax.experimental import pallas as pl
from jax.experimental.pallas import tpu as pltpu
from jax.experimental.pallas import tpu_sc as plsc

N = 100000
E = 1600000
H = 64
G = 64
IN = 4

NPAD = 100352
ZROWS = NPAD - N
TSLICE = NPAD // 16

EPAD = 1638400
EROWS = EPAD // 128
WROWS = EROWS // 32
BLKR = 4
NBLK = WROWS // BLKR

BR = 1792
NBR = NPAD // BR

_mesh = plsc.VectorSubcoreMesh(core_axis_name="c", subcore_axis_name="s")
_sc_params = pltpu.CompilerParams(use_tc_tiling_on_sc=False)


def _wid():
    return lax.axis_index("s") * 2 + lax.axis_index("c")


@functools.partial(
    pl.kernel,
    out_type=jax.ShapeDtypeStruct((2 * NPAD, 16), jnp.float32),
    mesh=_mesh,
    compiler_params=_sc_params,
    scratch_types=[
        pltpu.VMEM((BLKR, 128), jnp.int32),
        pltpu.VMEM((128, 16), jnp.float32),
        pltpu.VMEM_SHARED((NPAD, 16), jnp.float32),
    ],
)
def _deg_kernel(dstr_h, erow_h, zeros_h, out_h, didx_v, erow_v, acc_sh):
    cid = lax.axis_index("c")
    sid = lax.axis_index("s")
    wid = _wid()
    pltpu.sync_copy(erow_h, erow_v)
    pltpu.sync_copy(zeros_h, acc_sh.at[pl.ds(sid * TSLICE, TSLICE)])
    plsc.subcore_barrier()

    def blk(k, carry):
        base = wid * WROWS + k * BLKR
        pltpu.sync_copy(dstr_h.at[pl.ds(base, BLKR)], didx_v)
        for j in range(BLKR):
            pltpu.sync_copy(erow_v, acc_sh.at[didx_v.at[j]], add=True)
        return carry

    lax.fori_loop(0, NBLK, blk, 0)
    plsc.subcore_barrier()
    pltpu.sync_copy(
        acc_sh.at[pl.ds(sid * TSLICE, TSLICE)],
        out_h.at[pl.ds(cid * NPAD + sid * TSLICE, TSLICE)],
    )


def _make_agg(F, nsrc):
    @functools.partial(
        pl.kernel,
        out_type=jax.ShapeDtypeStruct((nsrc * 2 * NPAD, F), jnp.float32),
        mesh=_mesh,
        compiler_params=_sc_params,
        scratch_types=[
            pltpu.VMEM((2, BLKR, 128), jnp.int32),
            pltpu.VMEM((2, BLKR, 128), jnp.int32),
            pltpu.VMEM((2, BLKR * 128, F), jnp.float32),
            pltpu.VMEM_SHARED((NPAD, F), jnp.float32),
            pltpu.SemaphoreType.DMA,
            pltpu.SemaphoreType.DMA,
        ],
    )
    def _agg(srcr_h, dstr_h, src_h, zeros_h, out_h,
             sidx_v, didx_v, rows_v, acc_sh, sem0, sem1):
        cid = lax.axis_index("c")
        sid = lax.axis_index("s")
        wid = _wid()
        sems = (sem0, sem1)

        def fetch(k, b):
            base = wid * WROWS + k * BLKR
            pltpu.sync_copy(srcr_h.at[pl.ds(base, BLKR)], sidx_v.at[b])
            pltpu.sync_copy(dstr_h.at[pl.ds(base, BLKR)], didx_v.at[b])

        for p in range(nsrc):
            src_p = src_h.at[pl.ds(p * NPAD, NPAD)]
            pltpu.sync_copy(zeros_h, acc_sh.at[pl.ds(sid * TSLICE, TSLICE)])
            plsc.subcore_barrier()

            def fire(b):
                for j in range(BLKR):
                    pltpu.async_copy(
                        src_p.at[sidx_v.at[b, j]],
                        rows_v.at[b, pl.ds(j * 128, 128)],
                        sems[b],
                    )

            def drain_scatter(b):
                for j in range(BLKR):
                    pltpu.make_async_copy(
                        src_p.at[sidx_v.at[b, j]],
                        rows_v.at[b, pl.ds(j * 128, 128)],
                        sems[b],
                    ).wait()
                for j in range(BLKR):
                    pltpu.sync_copy(
                        rows_v.at[b, pl.ds(j * 128, 128)],
                        acc_sh.at[didx_v.at[b, j]],
                        add=True,
                    )

            fetch(0, 0)
            fire(0)

            def blk2(t, carry):
                fetch(2 * t + 1, 1)
                fire(1)
                drain_scatter(0)

                @pl.when(t < NBLK // 2 - 1)
                def _():
                    fetch(2 * t + 2, 0)
                    fire(0)

                drain_scatter(1)
                return carry

            lax.fori_loop(0, NBLK // 2, blk2, 0)
            plsc.subcore_barrier()
            pltpu.sync_copy(
                acc_sh.at[pl.ds(sid * TSLICE, TSLICE)],
                out_h.at[pl.ds((p * 2 + cid) * NPAD + sid * TSLICE, TSLICE)],
            )

    return _agg


_agg_in = _make_agg(16, 1)
_agg_h = _make_agg(16, 4)


NP8 = NPAD // 8
BRP = 1568
NBP = NP8 // BRP

_CON = (((1,), (0,)), ((), ()))


def _prep_body(deg0, deg1, x, dinv_o, xs_o):
    dinv = lax.rsqrt(deg0[...] + deg1[...] + 1.0)
    dinv_o[...] = dinv
    xs_o[...] = x[...] * dinv


def _prep_call(degp, xp):
    blk = pl.BlockSpec((BRP, 128), lambda i: (i, 0))
    return pl.pallas_call(
        _prep_body,
        grid=(NBP,),
        in_specs=[blk, pl.BlockSpec((BRP, 128), lambda i: (NBP + i, 0)), blk],
        out_specs=[blk, blk],
        out_shape=[jax.ShapeDtypeStruct((NP8, 128), jnp.float32)] * 2,
    )(degp, degp, xp)


def _dense1_body(a0, a1, xs, dinv, w, c, o):
    dv = dinv[...]
    a = (a0[...] + a1[...] + xs[...]) * dv
    z = lax.dot_general(a, w[...], _CON,
                        preferred_element_type=jnp.float32) + c[0]
    o[...] = jnp.maximum(z, 0.0) * dv


def _dense1_call(agg1p, xsp, dinvp, Ws1s, cs1s):
    blk = pl.BlockSpec((BRP, 128), lambda p, i: (i, 0))
    return pl.pallas_call(
        _dense1_body,
        grid=(4, NBP),
        in_specs=[
            blk,
            pl.BlockSpec((BRP, 128), lambda p, i: (NBP + i, 0)),
            blk, blk,
            pl.BlockSpec((128, 128), lambda p, i: (p, 0)),
            pl.BlockSpec((1, 1, 128), lambda p, i: (p, 0, 0)),
        ],
        out_specs=pl.BlockSpec((BRP, 128), lambda p, i: (p * NBP + i, 0)),
        out_shape=jax.ShapeDtypeStruct((4 * NP8, 128), jnp.float32),
    )(agg1p, agg1p, xsp, dinvp, Ws1s, cs1s)


def _dense2a_body(a00, a01, a10, a11, a20, a21, a30, a31,
                  h0, h1, h2, h3, dinv, w0, w1, w2, w3, bt2, wt3, o):
    dv = dinv[...]
    z = bt2[...]
    for ac, bc, hc, w in ((a00, a01, h0, w0), (a10, a11, h1, w1),
                          (a20, a21, h2, w2), (a30, a31, h3, w3)):
        a = (ac[...] + bc[...] + hc[...]) * dv
        z = z + lax.dot_general(a, w[...], _CON,
                                preferred_element_type=jnp.float32)
    hh = jnp.maximum(z, 0.0)
    o[...] = lax.dot_general(hh, wt3[...], _CON,
                             preferred_element_type=jnp.float32)


def _dense2a_call(agg2p, hstk, dinvp, Ws, bt2, Wt3):
    def off(k):
        return pl.BlockSpec((BRP, 128), lambda i, k=k: (k * NBP + i, 0))
    return pl.pallas_call(
        _dense2a_body,
        grid=(NBP,),
        in_specs=(
            [off(k) for k in range(8)]
            + [off(p) for p in range(4)]
            + [pl.BlockSpec((BRP, 128), lambda i: (i, 0))]
            + [pl.BlockSpec((128, 8 * H), lambda i: (0, 0))] * 4
            + [pl.BlockSpec((1, 8 * H), lambda i: (0, 0)),
               pl.BlockSpec((8 * H, 8), lambda i: (0, 0))]
        ),
        out_specs=pl.BlockSpec((BRP, 8), lambda i: (i, 0)),
        out_shape=jax.ShapeDtypeStruct((NP8, 8), jnp.float32),
    )(*([agg2p] * 8), *([hstk] * 4), dinvp, *Ws, bt2, Wt3)


BR8 = 1568
NB8 = NP8 // BR8


def _dense2b_body(s8, bat8, b3, out_o, accs_v, accc_v):
    g = pl.program_id(0)
    iota = lax.broadcasted_iota(jnp.int32, (1, G), 1)
    ps = jnp.zeros((G, 1), jnp.float32)
    pc = jnp.zeros((G, 1), jnp.float32)
    ones = jnp.ones((BR8, 1), jnp.float32)
    for j in range(8):
        ohj = (bat8[:, j:j + 1] == iota).astype(jnp.float32)
        ps = ps + lax.dot_general(ohj, s8[:, j:j + 1],
                                  (((0,), (0,)), ((), ())),
                                  preferred_element_type=jnp.float32)
        pc = pc + lax.dot_general(ohj, ones, (((0,), (0,)), ((), ())),
                                  preferred_element_type=jnp.float32)

    @pl.when(g == 0)
    def _():
        accs_v[...] = jnp.zeros((G, 1), jnp.float32)
        accc_v[...] = jnp.zeros((G, 1), jnp.float32)

    accs_v[...] += ps
    accc_v[...] += pc

    @pl.when(g == NB8 - 1)
    def _():
        out_o[...] = accs_v[...] / jnp.maximum(accc_v[...], 1.0) + b3[...]


def _dense2b_call(s8, bat8, b3):
    blk = pl.BlockSpec((BR8, 8), lambda i: (i, 0))
    return pl.pallas_call(
        _dense2b_body,
        grid=(NB8,),
        in_specs=[blk, blk, pl.BlockSpec((1, 1), lambda i: (0, 0))],
        out_specs=pl.BlockSpec((G, 1), lambda i: (0, 0)),
        out_shape=jax.ShapeDtypeStruct((G, 1), jnp.float32),
        scratch_shapes=[
            pltpu.VMEM((G, 1), jnp.float32),
            pltpu.VMEM((G, 1), jnp.float32),
        ],
    )(s8, bat8, b3)


def kernel(x, edge_index, batch, W1, b1, W2, b2, W3, b3):
    pad = N + (jnp.arange(EPAD - E, dtype=jnp.int32) % ZROWS)
    srcr = jnp.concatenate([edge_index[0], pad]).reshape(EROWS, 128)
    dstr = jnp.concatenate([edge_index[1], pad]).reshape(EROWS, 128)

    zeros16 = jnp.zeros((TSLICE, 16), jnp.float32)
    erow = jnp.ones((128, 16), jnp.float32)
    degp = _deg_kernel(dstr, erow, zeros16).reshape(2 * NP8, 128)

    xp = jnp.pad(x, ((0, NPAD - N), (0, 16 - IN))).reshape(NP8, 128)
    dinvp, xsp = _prep_call(degp, xp)

    agg1p = _agg_in(srcr, dstr, xsp.reshape(NPAD, 16),
                    zeros16).reshape(2 * NP8, 128)

    eye8 = jnp.eye(8, dtype=jnp.float32)
    W1p = jnp.pad(W1, ((0, 16 - IN), (0, 0)))
    Ws1s = jnp.concatenate(
        [jnp.kron(eye8, W1p[:, 16 * p:16 * p + 16]) for p in range(4)], axis=0)
    cs1s = jnp.stack(
        [jnp.tile(b1[16 * p:16 * p + 16], 8) for p in range(4)],
        axis=0).reshape(4, 1, 128)
    hstk = _dense1_call(agg1p, xsp, dinvp, Ws1s, cs1s)

    agg2p = _agg_h(srcr, dstr, hstk.reshape(4 * NPAD, 16),
                   zeros16).reshape(8 * NP8, 128)

    Ws2 = [jnp.kron(eye8, W2[16 * p:16 * p + 16, :]) for p in range(4)]
    bt2 = jnp.tile(b2, 8).reshape(1, 8 * H)
    Wt3 = jnp.kron(eye8, W3)
    sp = _dense2a_call(agg2p, hstk, dinvp, Ws2, bt2, Wt3)

    bat8 = jnp.pad(batch, (0, NPAD - N), constant_values=G).reshape(NP8, 8)
    return _dense2b_call(sp, bat8, b3.reshape(1, 1))
